# baseline (speedup 1.0000x reference)
import functools

import jax
import jax.numpy as jnp
from jax import lax
from jax.experimental import pallas as pl
from jax.experimental.pallas import tpu as pltpu
from jax.experimental.pallas import tpu_sc as plsc

N_E = 8192
D = 256
N_TOK = 8192
BETA = 0.25
EPS = 1e-10

ROW_BLK = 512
ROW_GRID = N_TOK // ROW_BLK
_WINDOWS = ((0, 2048), (2048, 2048), (4096, 2048), (6144, 2048))


def _dist_argmin_body(x_ref, e_ref, rowsq_ref, esq_ref, idx_ref, dsel_ref):
    sim2T = lax.dot_general(
        e_ref[...], x_ref[...] + x_ref[...],
        dimension_numbers=(((1,), (1,)), ((), ())),
        preferred_element_type=jnp.float32,
    )
    dT = (rowsq_ref[...][None, :] + esq_ref[...][:, None]) - sim2T
    accv = jnp.full((ROW_BLK,), jnp.inf, jnp.float32)
    accd = jnp.zeros((ROW_BLK,), jnp.float32)
    acci = jnp.zeros((ROW_BLK,), jnp.int32)
    for start, width in _WINDOWS:
        sl = dT[start:start + width, :]
        wv = jnp.min(sl, axis=0)
        wi = jnp.argmin(sl, axis=0).astype(jnp.int32) + start
        take = wv < accv
        accv = jnp.where(take, wv.astype(jnp.bfloat16).astype(jnp.float32), accv)
        accd = jnp.where(take, wv, accd)
        acci = jnp.where(take, wi, acci)
    idx_ref[...] = acci
    dsel_ref[...] = accd


_dist_argmin = pl.pallas_call(
    _dist_argmin_body,
    grid=(ROW_GRID,),
    in_specs=[
        pl.BlockSpec((ROW_BLK, D), lambda i: (i, 0)),
        pl.BlockSpec((N_E, D), lambda i: (0, 0)),
        pl.BlockSpec((ROW_BLK,), lambda i: (i,)),
        pl.BlockSpec((N_E,), lambda i: (0,)),
    ],
    out_specs=[
        pl.BlockSpec((ROW_BLK,), lambda i: (i,)),
        pl.BlockSpec((ROW_BLK,), lambda i: (i,)),
    ],
    out_shape=[
        jax.ShapeDtypeStruct((N_TOK,), jnp.int32),
        jax.ShapeDtypeStruct((N_TOK,), jnp.float32),
    ],
)


NW = 32
BPW = N_TOK // NW
_LANES = 16


@functools.lru_cache(maxsize=1)
def _make_sc_gather_count():
    info = plsc.get_sparse_core_info()
    nc, ns = info.num_cores, info.num_subcores
    assert nc * ns == NW

    @functools.partial(
        pl.kernel,
        mesh=plsc.VectorSubcoreMesh(core_axis_name="c", subcore_axis_name="s"),
        out_type=[
            jax.ShapeDtypeStruct((N_TOK, D), jnp.float32),
            jax.ShapeDtypeStruct((NW, N_E), jnp.int32),
        ],
        scratch_types=[
            pltpu.VMEM((BPW,), jnp.int32),
            pltpu.VMEM((BPW, D), jnp.float32),
            pltpu.VMEM((N_E,), jnp.int32),
            pltpu.SemaphoreType.DMA,
        ],
        compiler_params=pltpu.CompilerParams(needs_layout_passes=False),
    )
    def _sc_gather_count(emb_hbm, idx_hbm, z_hbm, q_hbm, pcnt_hbm,
                         idx_v, rows_v, cnt_v, sem):
        wid = lax.axis_index("s") * nc + lax.axis_index("c")
        base = wid * BPW
        pltpu.sync_copy(idx_hbm.at[pl.ds(base, BPW)], idx_v)
        gather = pltpu.async_copy(emb_hbm.at[idx_v], rows_v, sem)
        pltpu.sync_copy(z_hbm, cnt_v)

        ones = jnp.ones((_LANES,), jnp.int32)

        def _count(i, carry):
            vidx = idx_v[pl.ds(i * _LANES, _LANES)]
            plsc.addupdate_scatter(cnt_v, [vidx], ones)
            return carry

        lax.fori_loop(0, BPW // _LANES, _count, 0)

        gather.wait()
        pltpu.sync_copy(rows_v, q_hbm.at[pl.ds(base, BPW)])
        pltpu.sync_copy(cnt_v, pcnt_hbm.at[wid])

    return _sc_gather_count


def _stats_body(pcnt_ref, dmin_ref, cnt_ref, perp_ref, loss_ref):
    counts = jnp.sum(pcnt_ref[...], axis=0)
    cnt_ref[...] = counts
    p = counts.astype(jnp.float32) * (1.0 / N_TOK)
    ent = jnp.sum(p * jnp.log(p + EPS))
    perp_ref[0, 0] = jnp.exp(-ent)
    m = jnp.sum(dmin_ref[...]) * (1.0 / (N_TOK * D))
    loss_ref[0, 0] = BETA * m + m


_stats = pl.pallas_call(
    _stats_body,
    out_specs=[
        pl.BlockSpec(memory_space=pltpu.VMEM),
        pl.BlockSpec(memory_space=pltpu.SMEM),
        pl.BlockSpec(memory_space=pltpu.SMEM),
    ],
    out_shape=[
        jax.ShapeDtypeStruct((N_E,), jnp.int32),
        jax.ShapeDtypeStruct((1, 1), jnp.float32),
        jax.ShapeDtypeStruct((1, 1), jnp.float32),
    ],
)


def kernel(x, embeddings):
    flat = x.reshape(-1, D)
    rowsq = jnp.sum(flat ** 2, axis=1)
    esq = jnp.sum(embeddings ** 2, axis=1)
    idx, dmin = _dist_argmin(flat, embeddings, rowsq, esq)
    zeros = jnp.zeros((N_E,), jnp.int32)
    q, pcnt = _make_sc_gather_count()(embeddings, idx, zeros)
    counts, perp, loss = _stats(pcnt, dmin)
    quantized_ste = q.reshape(x.shape)
    return quantized_ste, perp[0, 0], loss[0, 0], counts


# speedup vs baseline: 1.0480x; 1.0480x over previous
import functools

import jax
import jax.numpy as jnp
from jax import lax
from jax.experimental import pallas as pl
from jax.experimental.pallas import tpu as pltpu
from jax.experimental.pallas import tpu_sc as plsc

N_E = 8192
D = 256
N_TOK = 8192
BETA = 0.25
EPS = 1e-10

ROW_BLK = 1024
ROW_GRID = N_TOK // ROW_BLK
_WINDOWS = ((0, 2048), (2048, 2048), (4096, 2048), (6144, 2048))


def _dist_argmin_body(x_ref, e_ref, rowsq_ref, esq_ref, idx_ref, dsel_ref):
    sim2T = lax.dot_general(
        e_ref[...], x_ref[...] + x_ref[...],
        dimension_numbers=(((1,), (1,)), ((), ())),
        preferred_element_type=jnp.float32,
    )
    dT = (rowsq_ref[...][None, :] + esq_ref[...][:, None]) - sim2T
    accv = jnp.full((ROW_BLK,), jnp.inf, jnp.float32)
    accd = jnp.zeros((ROW_BLK,), jnp.float32)
    acci = jnp.zeros((ROW_BLK,), jnp.int32)
    for start, width in _WINDOWS:
        sl = dT[start:start + width, :]
        wv = jnp.min(sl, axis=0)
        wi = jnp.argmin(sl, axis=0).astype(jnp.int32) + start
        take = wv < accv
        accv = jnp.where(take, wv.astype(jnp.bfloat16).astype(jnp.float32), accv)
        accd = jnp.where(take, wv, accd)
        acci = jnp.where(take, wi, acci)
    idx_ref[...] = acci
    dsel_ref[...] = accd


_dist_argmin = pl.pallas_call(
    _dist_argmin_body,
    grid=(ROW_GRID,),
    in_specs=[
        pl.BlockSpec((ROW_BLK, D), lambda i: (i, 0)),
        pl.BlockSpec((N_E, D), lambda i: (0, 0)),
        pl.BlockSpec((ROW_BLK,), lambda i: (i,)),
        pl.BlockSpec((N_E,), lambda i: (0,)),
    ],
    out_specs=[
        pl.BlockSpec((ROW_BLK,), lambda i: (i,)),
        pl.BlockSpec((ROW_BLK,), lambda i: (i,)),
    ],
    out_shape=[
        jax.ShapeDtypeStruct((N_TOK,), jnp.int32),
        jax.ShapeDtypeStruct((N_TOK,), jnp.float32),
    ],
    compiler_params=pltpu.CompilerParams(vmem_limit_bytes=120 * 1024 * 1024),
)


NW = 32
BPW = N_TOK // NW
_LANES = 16


@functools.lru_cache(maxsize=1)
def _make_sc_gather_count():
    info = plsc.get_sparse_core_info()
    nc, ns = info.num_cores, info.num_subcores
    assert nc * ns == NW

    @functools.partial(
        pl.kernel,
        mesh=plsc.VectorSubcoreMesh(core_axis_name="c", subcore_axis_name="s"),
        out_type=[
            jax.ShapeDtypeStruct((N_TOK, D), jnp.float32),
            jax.ShapeDtypeStruct((NW, N_E), jnp.int32),
        ],
        scratch_types=[
            pltpu.VMEM((BPW,), jnp.int32),
            pltpu.VMEM((BPW, D), jnp.float32),
            pltpu.VMEM((N_E,), jnp.int32),
            pltpu.SemaphoreType.DMA,
        ],
        compiler_params=pltpu.CompilerParams(needs_layout_passes=False),
    )
    def _sc_gather_count(emb_hbm, idx_hbm, q_hbm, pcnt_hbm,
                         idx_v, rows_v, cnt_v, sem):
        wid = lax.axis_index("s") * nc + lax.axis_index("c")
        base = wid * BPW
        pltpu.sync_copy(idx_hbm.at[pl.ds(base, BPW)], idx_v)
        gather = pltpu.async_copy(emb_hbm.at[idx_v], rows_v, sem)

        zeros = jnp.zeros((_LANES,), jnp.int32)

        def _zero(i, carry):
            cnt_v[pl.ds(i * _LANES, _LANES)] = zeros
            return carry

        lax.fori_loop(0, N_E // _LANES, _zero, 0)

        ones = jnp.ones((_LANES,), jnp.int32)

        def _count(i, carry):
            vidx = idx_v[pl.ds(i * _LANES, _LANES)]
            plsc.addupdate_scatter(cnt_v, [vidx], ones)
            return carry

        lax.fori_loop(0, BPW // _LANES, _count, 0)

        gather.wait()
        pltpu.sync_copy(rows_v, q_hbm.at[pl.ds(base, BPW)])
        pltpu.sync_copy(cnt_v, pcnt_hbm.at[wid])

    return _sc_gather_count


def _stats_body(pcnt_ref, dmin_ref, cnt_ref, perp_ref, loss_ref):
    counts = jnp.sum(pcnt_ref[...], axis=0)
    cnt_ref[...] = counts
    p = counts.astype(jnp.float32) * (1.0 / N_TOK)
    ent = jnp.sum(p * jnp.log(p + EPS))
    perp_ref[0, 0] = jnp.exp(-ent)
    m = jnp.sum(dmin_ref[...]) * (1.0 / (N_TOK * D))
    loss_ref[0, 0] = BETA * m + m


_stats = pl.pallas_call(
    _stats_body,
    out_specs=[
        pl.BlockSpec(memory_space=pltpu.VMEM),
        pl.BlockSpec(memory_space=pltpu.SMEM),
        pl.BlockSpec(memory_space=pltpu.SMEM),
    ],
    out_shape=[
        jax.ShapeDtypeStruct((N_E,), jnp.int32),
        jax.ShapeDtypeStruct((1, 1), jnp.float32),
        jax.ShapeDtypeStruct((1, 1), jnp.float32),
    ],
)


def kernel(x, embeddings):
    flat = x.reshape(-1, D)
    rowsq = jnp.sum(flat ** 2, axis=1)
    esq = jnp.sum(embeddings ** 2, axis=1)
    idx, dmin = _dist_argmin(flat, embeddings, rowsq, esq)
    q, pcnt = _make_sc_gather_count()(embeddings, idx)
    counts, perp, loss = _stats(pcnt, dmin)
    quantized_ste = q.reshape(x.shape)
    return quantized_ste, perp[0, 0], loss[0, 0], counts
